# Initial kernel scaffold; baseline (speedup 1.0000x reference)
import functools

import jax
import jax.numpy as jnp
from jax import lax
from jax.experimental import pallas as pl
from jax.experimental.pallas import tpu as pltpu
from jax.experimental.pallas import tpu_sc as plsc

N = 10000
E = 320000
H = 128

NC, NS = 2, 16
NW = NC * NS

_EPS = 1e-5


def _ln(h, g, b):
    m = jnp.mean(h, axis=-1, keepdims=True)
    v = jnp.mean((h - m) ** 2, axis=-1, keepdims=True)
    return (h - m) * lax.rsqrt(v + _EPS) * g + b


def _premul_body(x_ref, ws_ref, wd_ref, xs_ref, xd_ref):
    x = x_ref[...]
    xs_ref[...] = jnp.dot(x, ws_ref[...], preferred_element_type=jnp.float32)
    xd_ref[...] = jnp.dot(x, wd_ref[...], preferred_element_type=jnp.float32)


def _premul(x, ws, wd):
    bn = 2000
    return pl.pallas_call(
        _premul_body,
        grid=(N // bn,),
        in_specs=[
            pl.BlockSpec((bn, H), lambda i: (i, 0)),
            pl.BlockSpec((H, H), lambda i: (0, 0)),
            pl.BlockSpec((H, H), lambda i: (0, 0)),
        ],
        out_specs=[
            pl.BlockSpec((bn, H), lambda i: (i, 0)),
            pl.BlockSpec((bn, H), lambda i: (i, 0)),
        ],
        out_shape=[
            jax.ShapeDtypeStruct((N, H), jnp.float32),
            jax.ShapeDtypeStruct((N, H), jnp.float32),
        ],
    )(x, ws, wd)


_GCH = 128
_EPW = E // NW
_GN = -(-_EPW // _GCH)


def _gather_add(xs, xd, s_idx, r_idx):
    mesh = plsc.VectorSubcoreMesh(core_axis_name="c", subcore_axis_name="s")

    @functools.partial(
        pl.kernel,
        mesh=mesh,
        out_type=jax.ShapeDtypeStruct((E, H), jnp.float32),
        scratch_types=[
            pltpu.VMEM((_GCH,), jnp.int32),
            pltpu.VMEM((_GCH,), jnp.int32),
            pltpu.VMEM((_GCH, H), jnp.float32),
            pltpu.VMEM((_GCH, H), jnp.float32),
            pltpu.SemaphoreType.DMA,
            pltpu.SemaphoreType.DMA,
        ],
    )
    def k(xs_hbm, xd_hbm, s_hbm, r_hbm, g_hbm, sbuf, rbuf, rows_s, rows_r,
          sem1, sem2):
        wid = lax.axis_index("s") * NC + lax.axis_index("c")
        base0 = wid * _EPW

        def chunk(i, carry):
            off = lax.min(i * _GCH, _EPW - _GCH)
            base = base0 + off
            pltpu.sync_copy(s_hbm.at[pl.ds(base, _GCH)], sbuf)
            pltpu.sync_copy(r_hbm.at[pl.ds(base, _GCH)], rbuf)
            cp1 = pltpu.async_copy(xs_hbm.at[sbuf], rows_s, sem1)
            cp2 = pltpu.async_copy(xd_hbm.at[rbuf], rows_r, sem2)
            cp1.wait()
            cp2.wait()

            def add_row(rr, c2):
                for cc in range(H // 16):
                    sl = pl.ds(cc * 16, 16)
                    rows_s[rr, sl] = rows_s[rr, sl] + rows_r[rr, sl]
                return c2

            lax.fori_loop(0, _GCH, add_row, 0)
            pltpu.sync_copy(rows_s, g_hbm.at[pl.ds(base, _GCH), :])
            return carry

        lax.fori_loop(0, _GN, chunk, 0)

    return k(xs, xd, s_idx, r_idx)


def _edge_mlp_body(ea_ref, g_ref, a1_ref, b1_ref, w2_ref, b2_ref, w3_ref,
                   b3_ref, lng_ref, lnb_ref, eo_ref, en_ref):
    ea = ea_ref[...]
    h = jnp.dot(ea, a1_ref[...], preferred_element_type=jnp.float32)
    h = jax.nn.gelu(h + g_ref[...] + b1_ref[...], approximate=False)
    h = jnp.dot(h, w2_ref[...], preferred_element_type=jnp.float32)
    h = jax.nn.gelu(h + b2_ref[...], approximate=False)
    h = jnp.dot(h, w3_ref[...], preferred_element_type=jnp.float32) + b3_ref[...]
    en = _ln(h, lng_ref[...], lnb_ref[...])
    en_ref[...] = en
    eo_ref[...] = ea + en


def _edge_mlp(ea, g, a1, b1, w2, b2, w3, b3, lng, lnb):
    be = 1600
    row = pl.BlockSpec((be, H), lambda i: (i, 0))
    wspec = pl.BlockSpec((H, H), lambda i: (0, 0))
    vspec = pl.BlockSpec((1, H), lambda i: (0, 0))
    return pl.pallas_call(
        _edge_mlp_body,
        grid=(E // be,),
        in_specs=[row, row, wspec, vspec, wspec, vspec, wspec, vspec, vspec,
                  vspec],
        out_specs=[row, row],
        out_shape=[
            jax.ShapeDtypeStruct((E, H), jnp.float32),
            jax.ShapeDtypeStruct((E, H), jnp.float32),
        ],
    )(ea, g, a1, b1, w2, b2, w3, b3, lng, lnb)


_SCH = 80
_EPT = E // NS
_SN = _EPT // _SCH


def _scatter(en, ei_rs, zeros):
    mesh = plsc.VectorSubcoreMesh(core_axis_name="c", subcore_axis_name="s")

    @functools.partial(
        pl.kernel,
        mesh=mesh,
        out_type=jax.ShapeDtypeStruct((2, N, H), jnp.float32),
        scratch_types=[
            pltpu.VMEM((_SCH,), jnp.int32),
            pltpu.VMEM((_SCH, H), jnp.float32),
            pltpu.VMEM_SHARED((N, H), jnp.float32),
        ],
    )
    def k(en_hbm, ei_hbm, z_hbm, out_hbm, idxbuf, rows, agg):
        c = lax.axis_index("c")
        t = lax.axis_index("s")

        @pl.when(t == 0)
        def _():
            pltpu.sync_copy(z_hbm, agg)

        plsc.subcore_barrier()

        def chunk(i, carry):
            base = t * _EPT + i * _SCH
            pltpu.sync_copy(ei_hbm.at[c, pl.ds(base, _SCH)], idxbuf)
            pltpu.sync_copy(en_hbm.at[pl.ds(base, _SCH), :], rows)
            pltpu.sync_copy(rows, agg.at[idxbuf], add=True)
            return carry

        lax.fori_loop(0, _SN, chunk, 0)
        plsc.subcore_barrier()

        @pl.when(t == 0)
        def _():
            pltpu.sync_copy(agg, out_hbm.at[c])

    return k(en, ei_rs, zeros)


def _node_mlp_body(x_ref, ar_ref, as_ref, w1x_ref, w1r_ref, w1s_ref, b1_ref,
                   w2_ref, b2_ref, w3_ref, b3_ref, lng_ref, lnb_ref, xo_ref):
    x = x_ref[...]
    h = (jnp.dot(x, w1x_ref[...], preferred_element_type=jnp.float32)
         + jnp.dot(ar_ref[...], w1r_ref[...], preferred_element_type=jnp.float32)
         + jnp.dot(as_ref[...], w1s_ref[...], preferred_element_type=jnp.float32))
    h = jax.nn.gelu(h + b1_ref[...], approximate=False)
    h = jnp.dot(h, w2_ref[...], preferred_element_type=jnp.float32)
    h = jax.nn.gelu(h + b2_ref[...], approximate=False)
    h = jnp.dot(h, w3_ref[...], preferred_element_type=jnp.float32) + b3_ref[...]
    xo_ref[...] = x + _ln(h, lng_ref[...], lnb_ref[...])


def _node_mlp(x, aggr, aggs, w1x, w1r, w1s, b1, w2, b2, w3, b3, lng, lnb):
    bn = 2000
    row = pl.BlockSpec((bn, H), lambda i: (i, 0))
    wspec = pl.BlockSpec((H, H), lambda i: (0, 0))
    vspec = pl.BlockSpec((1, H), lambda i: (0, 0))
    return pl.pallas_call(
        _node_mlp_body,
        grid=(N // bn,),
        in_specs=[row, row, row, wspec, wspec, wspec, vspec, wspec, vspec,
                  wspec, vspec, vspec, vspec],
        out_specs=row,
        out_shape=jax.ShapeDtypeStruct((N, H), jnp.float32),
    )(x, aggr, aggs, w1x, w1r, w1s, b1, w2, b2, w3, b3, lng, lnb)


def kernel(x, edge_attr, edge_index, params):
    p = params
    senders = edge_index[0]
    receivers = edge_index[1]
    ei_rs = jnp.stack([receivers, senders])

    eb_w1 = p["eb_W1"]
    a1, ws, wd = eb_w1[:H], eb_w1[H:2 * H], eb_w1[2 * H:]

    xs1, xd1 = _premul(x, ws, wd)
    g = _gather_add(xs1, xd1, senders, receivers)

    r1 = lambda a: a.reshape(1, H)
    edge_out, edge_new = _edge_mlp(
        edge_attr, g, a1, r1(p["eb_b1"]), p["eb_W2"], r1(p["eb_b2"]),
        p["eb_W3"], r1(p["eb_b3"]), r1(p["eb_lng"]), r1(p["eb_lnb"]))

    aggp = _scatter(edge_new, ei_rs, jnp.zeros((N, H), jnp.float32))

    nb_w1 = p["nb_W1"]
    w1x = nb_w1[:H]
    half = nb_w1[H:]
    zpad = jnp.zeros((H // 2, H), jnp.float32)
    w1r = jnp.concatenate([half, zpad], axis=0)
    w1s = jnp.concatenate([zpad, half], axis=0)

    x_out = _node_mlp(
        x, aggp[0], aggp[1], w1x, w1r, w1s, r1(p["nb_b1"]), p["nb_W2"],
        r1(p["nb_b2"]), p["nb_W3"], r1(p["nb_b3"]), r1(p["nb_lng"]),
        r1(p["nb_lnb"]))

    return (x_out, edge_out)


# speedup vs baseline: 3.4557x; 3.4557x over previous
import functools

import jax
import jax.numpy as jnp
from jax import lax
from jax.experimental import pallas as pl
from jax.experimental.pallas import tpu as pltpu
from jax.experimental.pallas import tpu_sc as plsc

N = 10000
E = 320000
H = 128

NC, NS = 2, 16
NW = NC * NS

_EPS = 1e-5


def _gelu(h):
    return 0.5 * h * (1.0 + lax.erf(h * 0.7071067811865476))


def _ln(h, g, b):
    m = jnp.mean(h, axis=-1, keepdims=True)
    v = jnp.mean((h - m) ** 2, axis=-1, keepdims=True)
    return (h - m) * lax.rsqrt(v + _EPS) * g + b


def _premul_body(x_ref, ws_ref, wd_ref, xs_ref, xd_ref):
    x = x_ref[...]
    xs_ref[...] = jnp.dot(x, ws_ref[...], preferred_element_type=jnp.float32)
    xd_ref[...] = jnp.dot(x, wd_ref[...], preferred_element_type=jnp.float32)


def _premul(x, ws, wd):
    bn = 2000
    return pl.pallas_call(
        _premul_body,
        grid=(N // bn,),
        in_specs=[
            pl.BlockSpec((bn, H), lambda i: (i, 0)),
            pl.BlockSpec((H, H), lambda i: (0, 0)),
            pl.BlockSpec((H, H), lambda i: (0, 0)),
        ],
        out_specs=[
            pl.BlockSpec((bn, H), lambda i: (i, 0)),
            pl.BlockSpec((bn, H), lambda i: (i, 0)),
        ],
        out_shape=[
            jax.ShapeDtypeStruct((N, H), jnp.float32),
            jax.ShapeDtypeStruct((N, H), jnp.float32),
        ],
    )(x, ws, wd)


_GCH = 128
_EPW = E // NW
_GN = -(-_EPW // _GCH)


def _gather_add(xs, xd, s_idx, r_idx):
    mesh = plsc.VectorSubcoreMesh(core_axis_name="c", subcore_axis_name="s")

    @functools.partial(
        pl.kernel,
        mesh=mesh,
        out_type=jax.ShapeDtypeStruct((E, H), jnp.float32),
        scratch_types=[
            pltpu.VMEM((_GCH,), jnp.int32),
            pltpu.VMEM((_GCH,), jnp.int32),
            pltpu.VMEM((_GCH, H), jnp.float32),
            pltpu.VMEM((_GCH, H), jnp.float32),
            pltpu.SemaphoreType.DMA,
            pltpu.SemaphoreType.DMA,
        ],
    )
    def k(xs_hbm, xd_hbm, s_hbm, r_hbm, g_hbm, sbuf, rbuf, rows_s, rows_r,
          sem1, sem2):
        wid = lax.axis_index("s") * NC + lax.axis_index("c")
        base0 = wid * _EPW

        def chunk(i, carry):
            off = lax.min(i * _GCH, _EPW - _GCH)
            base = base0 + off
            pltpu.sync_copy(s_hbm.at[pl.ds(base, _GCH)], sbuf)
            pltpu.sync_copy(r_hbm.at[pl.ds(base, _GCH)], rbuf)
            cp1 = pltpu.async_copy(xs_hbm.at[sbuf], rows_s, sem1)
            cp2 = pltpu.async_copy(xd_hbm.at[rbuf], rows_r, sem2)
            cp1.wait()
            cp2.wait()

            def add_row(rr, c2):
                for cc in range(H // 16):
                    sl = pl.ds(cc * 16, 16)
                    rows_s[rr, sl] = rows_s[rr, sl] + rows_r[rr, sl]
                return c2

            lax.fori_loop(0, _GCH, add_row, 0)
            pltpu.sync_copy(rows_s, g_hbm.at[pl.ds(base, _GCH), :])
            return carry

        lax.fori_loop(0, _GN, chunk, 0)

    return k(xs, xd, s_idx, r_idx)


def _edge_mlp_body(ea_ref, g_ref, a1_ref, b1_ref, w2_ref, b2_ref, w3_ref,
                   b3_ref, lng_ref, lnb_ref, eo_ref, en_ref):
    ea = ea_ref[...]
    h = jnp.dot(ea, a1_ref[...], preferred_element_type=jnp.float32)
    h = _gelu(h + g_ref[...] + b1_ref[...])
    h = jnp.dot(h, w2_ref[...], preferred_element_type=jnp.float32)
    h = _gelu(h + b2_ref[...])
    h = jnp.dot(h, w3_ref[...], preferred_element_type=jnp.float32) + b3_ref[...]
    en = _ln(h, lng_ref[...], lnb_ref[...])
    en_ref[...] = en
    eo_ref[...] = ea + en


def _edge_mlp(ea, g, a1, b1, w2, b2, w3, b3, lng, lnb):
    be = 1600
    row = pl.BlockSpec((be, H), lambda i: (i, 0))
    wspec = pl.BlockSpec((H, H), lambda i: (0, 0))
    vspec = pl.BlockSpec((1, H), lambda i: (0, 0))
    return pl.pallas_call(
        _edge_mlp_body,
        grid=(E // be,),
        in_specs=[row, row, wspec, vspec, wspec, vspec, wspec, vspec, vspec,
                  vspec],
        out_specs=[row, row],
        out_shape=[
            jax.ShapeDtypeStruct((E, H), jnp.float32),
            jax.ShapeDtypeStruct((E, H), jnp.float32),
        ],
    )(ea, g, a1, b1, w2, b2, w3, b3, lng, lnb)


_SCH = 80
_EPT = E // NS
_SN = _EPT // _SCH


def _scatter(en, ridx, sidx, zeros):
    mesh = plsc.VectorSubcoreMesh(core_axis_name="c", subcore_axis_name="s")

    @functools.partial(
        pl.kernel,
        mesh=mesh,
        out_type=jax.ShapeDtypeStruct((2, N, H), jnp.float32),
        scratch_types=[
            pltpu.VMEM((_SCH,), jnp.int32),
            pltpu.VMEM((_SCH, H), jnp.float32),
            pltpu.VMEM_SHARED((N, H), jnp.float32),
        ],
    )
    def k(en_hbm, r_hbm, s_hbm, z_hbm, out_hbm, idxbuf, rows, agg):
        c = lax.axis_index("c")
        t = lax.axis_index("s")

        @pl.when(t == 0)
        def _():
            pltpu.sync_copy(z_hbm, agg)

        plsc.subcore_barrier()

        def chunk(i, carry):
            base = t * _EPT + i * _SCH

            @pl.when(c == 0)
            def _():
                pltpu.sync_copy(r_hbm.at[pl.ds(base, _SCH)], idxbuf)

            @pl.when(c == 1)
            def _():
                pltpu.sync_copy(s_hbm.at[pl.ds(base, _SCH)], idxbuf)

            pltpu.sync_copy(en_hbm.at[pl.ds(base, _SCH), :], rows)
            pltpu.sync_copy(rows, agg.at[idxbuf], add=True)
            return carry

        lax.fori_loop(0, _SN, chunk, 0)
        plsc.subcore_barrier()

        @pl.when(t == 0)
        def _():
            pltpu.sync_copy(agg, out_hbm.at[c])

    return k(en, ridx, sidx, zeros)


def _node_mlp_body(x_ref, ar_ref, as_ref, w1x_ref, w1r_ref, w1s_ref, b1_ref,
                   w2_ref, b2_ref, w3_ref, b3_ref, lng_ref, lnb_ref, xo_ref):
    x = x_ref[...]
    h = (jnp.dot(x, w1x_ref[...], preferred_element_type=jnp.float32)
         + jnp.dot(ar_ref[...], w1r_ref[...], preferred_element_type=jnp.float32)
         + jnp.dot(as_ref[...], w1s_ref[...], preferred_element_type=jnp.float32))
    h = _gelu(h + b1_ref[...])
    h = jnp.dot(h, w2_ref[...], preferred_element_type=jnp.float32)
    h = _gelu(h + b2_ref[...])
    h = jnp.dot(h, w3_ref[...], preferred_element_type=jnp.float32) + b3_ref[...]
    xo_ref[...] = x + _ln(h, lng_ref[...], lnb_ref[...])


def _node_mlp(x, aggr, aggs, w1x, w1r, w1s, b1, w2, b2, w3, b3, lng, lnb):
    bn = 2000
    row = pl.BlockSpec((bn, H), lambda i: (i, 0))
    wspec = pl.BlockSpec((H, H), lambda i: (0, 0))
    vspec = pl.BlockSpec((1, H), lambda i: (0, 0))
    return pl.pallas_call(
        _node_mlp_body,
        grid=(N // bn,),
        in_specs=[row, row, row, wspec, wspec, wspec, vspec, wspec, vspec,
                  wspec, vspec, vspec, vspec],
        out_specs=row,
        out_shape=jax.ShapeDtypeStruct((N, H), jnp.float32),
    )(x, aggr, aggs, w1x, w1r, w1s, b1, w2, b2, w3, b3, lng, lnb)


def kernel(x, edge_attr, edge_index, params):
    p = params
    senders = edge_index[0]
    receivers = edge_index[1]
    eb_w1 = p["eb_W1"]
    a1, ws, wd = eb_w1[:H], eb_w1[H:2 * H], eb_w1[2 * H:]

    xs1, xd1 = _premul(x, ws, wd)
    g = _gather_add(xs1, xd1, senders, receivers)

    r1 = lambda a: a.reshape(1, H)
    edge_out, edge_new = _edge_mlp(
        edge_attr, g, a1, r1(p["eb_b1"]), p["eb_W2"], r1(p["eb_b2"]),
        p["eb_W3"], r1(p["eb_b3"]), r1(p["eb_lng"]), r1(p["eb_lnb"]))

    aggp = _scatter(edge_new, receivers, senders,
                    jnp.zeros((N, H), jnp.float32))

    nb_w1 = p["nb_W1"]
    w1x = nb_w1[:H]
    half = nb_w1[H:]
    zpad = jnp.zeros((H // 2, H), jnp.float32)
    w1r = jnp.concatenate([half, zpad], axis=0)
    w1s = jnp.concatenate([zpad, half], axis=0)

    x_out = _node_mlp(
        x, aggp[0], aggp[1], w1x, w1r, w1s, r1(p["nb_b1"]), p["nb_W2"],
        r1(p["nb_b2"]), p["nb_W3"], r1(p["nb_b3"]), r1(p["nb_lng"]),
        r1(p["nb_lnb"]))

    return (x_out, edge_out)


# speedup vs baseline: 5.5549x; 1.6075x over previous
import functools

import jax
import jax.numpy as jnp
from jax import lax
from jax.experimental import pallas as pl
from jax.experimental.pallas import tpu as pltpu
from jax.experimental.pallas import tpu_sc as plsc

N = 10000
E = 320000
H = 128

NC, NS = 2, 16
NW = NC * NS

_EPS = 1e-5


def _gelu(h):
    return 0.5 * h * (1.0 + lax.erf(h * 0.7071067811865476))


def _ln(h, g, b):
    m = jnp.mean(h, axis=-1, keepdims=True)
    v = jnp.mean((h - m) ** 2, axis=-1, keepdims=True)
    return (h - m) * lax.rsqrt(v + _EPS) * g + b


def _premul_body(x_ref, ws_ref, wd_ref, xs_ref, xd_ref):
    x = x_ref[...]
    xs_ref[...] = jnp.dot(x, ws_ref[...], preferred_element_type=jnp.float32)
    xd_ref[...] = jnp.dot(x, wd_ref[...], preferred_element_type=jnp.float32)


def _premul(x, ws, wd):
    bn = 2000
    return pl.pallas_call(
        _premul_body,
        grid=(N // bn,),
        in_specs=[
            pl.BlockSpec((bn, H), lambda i: (i, 0)),
            pl.BlockSpec((H, H), lambda i: (0, 0)),
            pl.BlockSpec((H, H), lambda i: (0, 0)),
        ],
        out_specs=[
            pl.BlockSpec((bn, H), lambda i: (i, 0)),
            pl.BlockSpec((bn, H), lambda i: (i, 0)),
        ],
        out_shape=[
            jax.ShapeDtypeStruct((N, H), jnp.float32),
            jax.ShapeDtypeStruct((N, H), jnp.float32),
        ],
    )(x, ws, wd)


_GCH = 128
_EPW = E // NW
_GN = 2 * (-(-_EPW // (2 * _GCH)))


def _gather_add(xs, xd, s_idx, r_idx):
    mesh = plsc.VectorSubcoreMesh(core_axis_name="c", subcore_axis_name="s")

    @functools.partial(
        pl.kernel,
        mesh=mesh,
        out_type=jax.ShapeDtypeStruct((E, H), jnp.float32),
        scratch_types=[
            pltpu.VMEM((_EPW,), jnp.int32),
            pltpu.VMEM((_EPW,), jnp.int32),
            [pltpu.VMEM((_GCH, H), jnp.float32)] * 2,
            [pltpu.VMEM((_GCH, H), jnp.float32)] * 2,
            [pltpu.SemaphoreType.DMA] * 2,
            [pltpu.SemaphoreType.DMA] * 2,
        ],
    )
    def k(xs_hbm, xd_hbm, s_hbm, r_hbm, g_hbm, sall, rall, rows_s, rows_r,
          sem_s, sem_r):
        wid = lax.axis_index("s") * NC + lax.axis_index("c")
        base0 = wid * _EPW

        pltpu.sync_copy(s_hbm.at[pl.ds(base0, _EPW)], sall)
        pltpu.sync_copy(r_hbm.at[pl.ds(base0, _EPW)], rall)

        def off_of(j):
            return lax.min(j * _GCH, _EPW - _GCH)

        def fire(j, b):
            off = off_of(j)
            pltpu.async_copy(xs_hbm.at[sall.at[pl.ds(off, _GCH)]],
                             rows_s[b], sem_s[b])
            pltpu.async_copy(xd_hbm.at[rall.at[pl.ds(off, _GCH)]],
                             rows_r[b], sem_r[b])

        def drain(b):
            pltpu.make_async_copy(xs_hbm.at[sall.at[pl.ds(0, _GCH)]],
                                  rows_s[b], sem_s[b]).wait()
            pltpu.make_async_copy(xd_hbm.at[rall.at[pl.ds(0, _GCH)]],
                                  rows_r[b], sem_r[b]).wait()

        fire(0, 0)

        def outer(i, carry):
            for b in range(2):
                j = 2 * i + b

                @pl.when(j + 1 < _GN)
                def _():
                    fire(j + 1, 1 - b)

                drain(b)

                def add_row(rr, c2):
                    for cc in range(H // 16):
                        sl = pl.ds(cc * 16, 16)
                        rows_s[b][rr, sl] = rows_s[b][rr, sl] + rows_r[b][rr, sl]
                    return c2

                lax.fori_loop(0, _GCH, add_row, 0)
                pltpu.sync_copy(rows_s[b],
                                g_hbm.at[pl.ds(base0 + off_of(j), _GCH), :])
            return carry

        lax.fori_loop(0, _GN // 2, outer, 0)

    return k(xs, xd, s_idx, r_idx)


def _edge_mlp_body(ea_ref, g_ref, a1_ref, b1_ref, w2_ref, b2_ref, w3_ref,
                   b3_ref, lng_ref, lnb_ref, eo_ref, en_ref):
    ea = ea_ref[...]
    h = jnp.dot(ea, a1_ref[...], preferred_element_type=jnp.float32)
    h = _gelu(h + g_ref[...] + b1_ref[...])
    h = jnp.dot(h, w2_ref[...], preferred_element_type=jnp.float32)
    h = _gelu(h + b2_ref[...])
    h = jnp.dot(h, w3_ref[...], preferred_element_type=jnp.float32) + b3_ref[...]
    en = _ln(h, lng_ref[...], lnb_ref[...])
    en_ref[...] = en
    eo_ref[...] = ea + en


def _edge_mlp(ea, g, a1, b1, w2, b2, w3, b3, lng, lnb):
    be = 1600
    row = pl.BlockSpec((be, H), lambda i: (i, 0))
    wspec = pl.BlockSpec((H, H), lambda i: (0, 0))
    vspec = pl.BlockSpec((1, H), lambda i: (0, 0))
    return pl.pallas_call(
        _edge_mlp_body,
        grid=(E // be,),
        in_specs=[row, row, wspec, vspec, wspec, vspec, wspec, vspec, vspec,
                  vspec],
        out_specs=[row, row],
        out_shape=[
            jax.ShapeDtypeStruct((E, H), jnp.float32),
            jax.ShapeDtypeStruct((E, H), jnp.float32),
        ],
    )(ea, g, a1, b1, w2, b2, w3, b3, lng, lnb)


_SCH = 80
_EPT = E // NS
_SN = _EPT // _SCH


def _scatter(en, ridx, sidx, zeros):
    mesh = plsc.VectorSubcoreMesh(core_axis_name="c", subcore_axis_name="s")

    @functools.partial(
        pl.kernel,
        mesh=mesh,
        out_type=jax.ShapeDtypeStruct((2, N, H), jnp.float32),
        scratch_types=[
            [pltpu.VMEM((_SCH,), jnp.int32)] * 2,
            [pltpu.VMEM((_SCH, H), jnp.float32)] * 2,
            pltpu.VMEM_SHARED((N, H), jnp.float32),
            [pltpu.SemaphoreType.DMA] * 2,
            [pltpu.SemaphoreType.DMA] * 2,
        ],
    )
    def k(en_hbm, r_hbm, s_hbm, z_hbm, out_hbm, idxbuf, rows, agg, sem_i,
          sem_d):
        c = lax.axis_index("c")
        t = lax.axis_index("s")

        @pl.when(t == 0)
        def _():
            pltpu.sync_copy(z_hbm, agg)

        plsc.subcore_barrier()

        def fire(j, b):
            base = t * _EPT + j * _SCH

            @pl.when(c == 0)
            def _():
                pltpu.async_copy(r_hbm.at[pl.ds(base, _SCH)], idxbuf[b],
                                 sem_i[b])

            @pl.when(c == 1)
            def _():
                pltpu.async_copy(s_hbm.at[pl.ds(base, _SCH)], idxbuf[b],
                                 sem_i[b])

            pltpu.async_copy(en_hbm.at[pl.ds(base, _SCH), :], rows[b],
                             sem_d[b])

        def drain(b):
            pltpu.make_async_copy(r_hbm.at[pl.ds(0, _SCH)], idxbuf[b],
                                  sem_i[b]).wait()
            pltpu.make_async_copy(en_hbm.at[pl.ds(0, _SCH), :], rows[b],
                                  sem_d[b]).wait()

        fire(0, 0)

        def outer(i, carry):
            for b in range(2):
                j = 2 * i + b

                @pl.when(j + 1 < _SN)
                def _():
                    fire(j + 1, 1 - b)

                drain(b)
                pltpu.sync_copy(rows[b], agg.at[idxbuf[b]], add=True)
            return carry

        lax.fori_loop(0, _SN // 2, outer, 0)
        plsc.subcore_barrier()

        @pl.when(t == 0)
        def _():
            pltpu.sync_copy(agg, out_hbm.at[c])

    return k(en, ridx, sidx, zeros)


def _node_mlp_body(x_ref, ar_ref, as_ref, w1x_ref, w1r_ref, w1s_ref, b1_ref,
                   w2_ref, b2_ref, w3_ref, b3_ref, lng_ref, lnb_ref, xo_ref):
    x = x_ref[...]
    h = (jnp.dot(x, w1x_ref[...], preferred_element_type=jnp.float32)
         + jnp.dot(ar_ref[...], w1r_ref[...], preferred_element_type=jnp.float32)
         + jnp.dot(as_ref[...], w1s_ref[...], preferred_element_type=jnp.float32))
    h = _gelu(h + b1_ref[...])
    h = jnp.dot(h, w2_ref[...], preferred_element_type=jnp.float32)
    h = _gelu(h + b2_ref[...])
    h = jnp.dot(h, w3_ref[...], preferred_element_type=jnp.float32) + b3_ref[...]
    xo_ref[...] = x + _ln(h, lng_ref[...], lnb_ref[...])


def _node_mlp(x, aggr, aggs, w1x, w1r, w1s, b1, w2, b2, w3, b3, lng, lnb):
    bn = 2000
    row = pl.BlockSpec((bn, H), lambda i: (i, 0))
    wspec = pl.BlockSpec((H, H), lambda i: (0, 0))
    vspec = pl.BlockSpec((1, H), lambda i: (0, 0))
    return pl.pallas_call(
        _node_mlp_body,
        grid=(N // bn,),
        in_specs=[row, row, row, wspec, wspec, wspec, vspec, wspec, vspec,
                  wspec, vspec, vspec, vspec],
        out_specs=row,
        out_shape=jax.ShapeDtypeStruct((N, H), jnp.float32),
    )(x, aggr, aggs, w1x, w1r, w1s, b1, w2, b2, w3, b3, lng, lnb)


def kernel(x, edge_attr, edge_index, params):
    p = params
    senders = edge_index[0]
    receivers = edge_index[1]
    eb_w1 = p["eb_W1"]
    a1, ws, wd = eb_w1[:H], eb_w1[H:2 * H], eb_w1[2 * H:]

    xs1, xd1 = _premul(x, ws, wd)
    g = _gather_add(xs1, xd1, senders, receivers)

    r1 = lambda a: a.reshape(1, H)
    edge_out, edge_new = _edge_mlp(
        edge_attr, g, a1, r1(p["eb_b1"]), p["eb_W2"], r1(p["eb_b2"]),
        p["eb_W3"], r1(p["eb_b3"]), r1(p["eb_lng"]), r1(p["eb_lnb"]))

    aggp = _scatter(edge_new, receivers, senders,
                    jnp.zeros((N, H), jnp.float32))

    nb_w1 = p["nb_W1"]
    w1x = nb_w1[:H]
    half = nb_w1[H:]
    zpad = jnp.zeros((H // 2, H), jnp.float32)
    w1r = jnp.concatenate([half, zpad], axis=0)
    w1s = jnp.concatenate([zpad, half], axis=0)

    x_out = _node_mlp(
        x, aggp[0], aggp[1], w1x, w1r, w1s, r1(p["nb_b1"]), p["nb_W2"],
        r1(p["nb_b2"]), p["nb_W3"], r1(p["nb_b3"]), r1(p["nb_lng"]),
        r1(p["nb_lnb"]))

    return (x_out, edge_out)


# speedup vs baseline: 6.0739x; 1.0934x over previous
import functools

import jax
import jax.numpy as jnp
from jax import lax
from jax.experimental import pallas as pl
from jax.experimental.pallas import tpu as pltpu
from jax.experimental.pallas import tpu_sc as plsc

N = 10000
E = 320000
H = 128

NC, NS = 2, 16
NW = NC * NS

_EPS = 1e-5

_NP = 5
_EP = E // _NP


def _gelu(h):
    return 0.5 * h * (1.0 + lax.erf(h * 0.7071067811865476))


def _ln(h, g, b):
    m = jnp.mean(h, axis=-1, keepdims=True)
    v = jnp.mean((h - m) ** 2, axis=-1, keepdims=True)
    return (h - m) * lax.rsqrt(v + _EPS) * g + b


def _premul_body(x_ref, ws_ref, wd_ref, xs_ref, xd_ref):
    x = x_ref[...]
    xs_ref[...] = jnp.dot(x, ws_ref[...], preferred_element_type=jnp.float32)
    xd_ref[...] = jnp.dot(x, wd_ref[...], preferred_element_type=jnp.float32)


def _premul(x, ws, wd):
    bn = 2000
    return pl.pallas_call(
        _premul_body,
        grid=(N // bn,),
        in_specs=[
            pl.BlockSpec((bn, H), lambda i: (i, 0)),
            pl.BlockSpec((H, H), lambda i: (0, 0)),
            pl.BlockSpec((H, H), lambda i: (0, 0)),
        ],
        out_specs=[
            pl.BlockSpec((bn, H), lambda i: (i, 0)),
            pl.BlockSpec((bn, H), lambda i: (i, 0)),
        ],
        out_shape=[
            jax.ShapeDtypeStruct((N, H), jnp.float32),
            jax.ShapeDtypeStruct((N, H), jnp.float32),
        ],
    )(x, ws, wd)


_GCH = 128


def _gather_add(xs, xd, s_idx, r_idx):
    mesh = plsc.VectorSubcoreMesh(core_axis_name="c", subcore_axis_name="s")
    ep = s_idx.shape[0]
    epw = ep // NW
    gn = -(-epw // _GCH)

    @functools.partial(
        pl.kernel,
        mesh=mesh,
        out_type=jax.ShapeDtypeStruct((ep, H), jnp.float32),
        scratch_types=[
            pltpu.VMEM((epw,), jnp.int32),
            pltpu.VMEM((epw,), jnp.int32),
            [pltpu.VMEM((_GCH, H), jnp.float32)] * 2,
            [pltpu.VMEM((_GCH, H), jnp.float32)] * 2,
            [pltpu.SemaphoreType.DMA] * 2,
            [pltpu.SemaphoreType.DMA] * 2,
        ],
    )
    def k(xs_hbm, xd_hbm, s_hbm, r_hbm, g_hbm, sall, rall, rows_s, rows_r,
          sem_s, sem_r):
        wid = lax.axis_index("s") * NC + lax.axis_index("c")
        base0 = wid * epw

        pltpu.sync_copy(s_hbm.at[pl.ds(base0, epw)], sall)
        pltpu.sync_copy(r_hbm.at[pl.ds(base0, epw)], rall)

        def off_of(j):
            return lax.min(j * _GCH, epw - _GCH)

        def fire(j, b):
            off = off_of(j)
            pltpu.async_copy(xs_hbm.at[sall.at[pl.ds(off, _GCH)]],
                             rows_s[b], sem_s[b])
            pltpu.async_copy(xd_hbm.at[rall.at[pl.ds(off, _GCH)]],
                             rows_r[b], sem_r[b])

        def proc(j, b):
            pltpu.make_async_copy(xs_hbm.at[sall.at[pl.ds(0, _GCH)]],
                                  rows_s[b], sem_s[b]).wait()
            pltpu.make_async_copy(xd_hbm.at[rall.at[pl.ds(0, _GCH)]],
                                  rows_r[b], sem_r[b]).wait()

            def add_row(rr, c2):
                for cc in range(H // 16):
                    sl = pl.ds(cc * 16, 16)
                    rows_s[b][rr, sl] = rows_s[b][rr, sl] + rows_r[b][rr, sl]
                return c2

            lax.fori_loop(0, _GCH, add_row, 0)
            pltpu.sync_copy(rows_s[b],
                            g_hbm.at[pl.ds(base0 + off_of(j), _GCH), :])

        fire(0, 0)
        npairs = (gn - 1) // 2

        def outer(i, carry):
            for b in range(2):
                j = 2 * i + b
                fire(j + 1, 1 - b)
                proc(j, b)
            return carry

        lax.fori_loop(0, npairs, outer, 0)
        for j in range(2 * npairs, gn):
            if j + 1 < gn:
                fire(j + 1, (j + 1) % 2)
            proc(j, j % 2)

    return k(xs, xd, s_idx, r_idx)


def _edge_mlp_body(ea_ref, g_ref, a1_ref, b1_ref, w2_ref, b2_ref, w3_ref,
                   b3_ref, lng_ref, lnb_ref, eo_ref, en_ref):
    ea = ea_ref[...]
    h = jnp.dot(ea, a1_ref[...], preferred_element_type=jnp.float32)
    h = _gelu(h + g_ref[...] + b1_ref[...])
    h = jnp.dot(h, w2_ref[...], preferred_element_type=jnp.float32)
    h = _gelu(h + b2_ref[...])
    h = jnp.dot(h, w3_ref[...], preferred_element_type=jnp.float32) + b3_ref[...]
    en = _ln(h, lng_ref[...], lnb_ref[...])
    en_ref[...] = en
    eo_ref[...] = ea + en


def _edge_mlp_body_aliased(ea_ref, g_ref, a1_ref, b1_ref, w2_ref, b2_ref,
                           w3_ref, b3_ref, lng_ref, lnb_ref, eoprev_ref,
                           eo_ref, en_ref):
    _edge_mlp_body(ea_ref, g_ref, a1_ref, b1_ref, w2_ref, b2_ref, w3_ref,
                   b3_ref, lng_ref, lnb_ref, eo_ref, en_ref)


def _edge_mlp(piece, ea, g, a1, b1, w2, b2, w3, b3, lng, lnb, eo_prev):
    be = 1600
    ep = g.shape[0]
    off = piece * ep // be
    rowg = pl.BlockSpec((be, H), lambda i: (i, 0))
    rowe = pl.BlockSpec((be, H), lambda i: (i + off, 0))
    wspec = pl.BlockSpec((H, H), lambda i: (0, 0))
    vspec = pl.BlockSpec((1, H), lambda i: (0, 0))
    in_specs = [rowe, rowg, wspec, vspec, wspec, vspec, wspec, vspec, vspec,
                vspec]
    args = [ea, g, a1, b1, w2, b2, w3, b3, lng, lnb]
    if eo_prev is None:
        body = _edge_mlp_body
        aliases = {}
    else:
        body = _edge_mlp_body_aliased
        in_specs = in_specs + [pl.BlockSpec((8, H), lambda i: (0, 0))]
        args = args + [eo_prev]
        aliases = {10: 0}
    return pl.pallas_call(
        body,
        grid=(ep // be,),
        in_specs=in_specs,
        out_specs=[rowe, rowg],
        out_shape=[
            jax.ShapeDtypeStruct((E, H), jnp.float32),
            jax.ShapeDtypeStruct((ep, H), jnp.float32),
        ],
        input_output_aliases=aliases,
    )(*args)


_SCH = 80


def _scatter(en, ridx, sidx, zeros, first=False):
    mesh = plsc.VectorSubcoreMesh(core_axis_name="c", subcore_axis_name="s")
    ep = en.shape[0]
    ept = ep // NS
    sn = ept // _SCH

    @functools.partial(
        pl.kernel,
        mesh=mesh,
        out_type=jax.ShapeDtypeStruct((2, N, H), jnp.float32),
        scratch_types=[
            [pltpu.VMEM((_SCH,), jnp.int32)] * 2,
            [pltpu.VMEM((_SCH, H), jnp.float32)] * 2,
            pltpu.VMEM_SHARED((N, H), jnp.float32),
            [pltpu.SemaphoreType.DMA] * 2,
            [pltpu.SemaphoreType.DMA] * 2,
        ],
    )
    def k(en_hbm, r_hbm, s_hbm, z_hbm, out_hbm, idxbuf, rows, agg, sem_i,
          sem_d):
        c = lax.axis_index("c")
        t = lax.axis_index("s")

        @pl.when(t == 0)
        def _():
            pltpu.sync_copy(z_hbm, agg)

        plsc.subcore_barrier()

        def fire(j, b):
            base = t * ept + j * _SCH

            @pl.when(c == 0)
            def _():
                pltpu.async_copy(r_hbm.at[pl.ds(base, _SCH)], idxbuf[b],
                                 sem_i[b])

            @pl.when(c == 1)
            def _():
                pltpu.async_copy(s_hbm.at[pl.ds(base, _SCH)], idxbuf[b],
                                 sem_i[b])

            pltpu.async_copy(en_hbm.at[pl.ds(base, _SCH), :], rows[b],
                             sem_d[b])

        def scat(b):
            pltpu.make_async_copy(r_hbm.at[pl.ds(0, _SCH)], idxbuf[b],
                                  sem_i[b]).wait()
            pltpu.make_async_copy(en_hbm.at[pl.ds(0, _SCH), :], rows[b],
                                  sem_d[b]).wait()
            pltpu.sync_copy(rows[b], agg.at[idxbuf[b]], add=True)

        fire(0, 0)
        npairs = (sn - 1) // 2

        def outer(i, carry):
            for b in range(2):
                j = 2 * i + b
                fire(j + 1, 1 - b)
                scat(b)
            return carry

        lax.fori_loop(0, npairs, outer, 0)
        for j in range(2 * npairs, sn):
            if j + 1 < sn:
                fire(j + 1, (j + 1) % 2)
            scat(j % 2)
        plsc.subcore_barrier()

        @pl.when(t == 0)
        def _():
            pltpu.sync_copy(agg, out_hbm.at[c])

    return k(en, ridx, sidx, zeros)


def _node_mlp_body(*refs):
    x_ref = refs[0]
    np_ = _NP
    ar_refs = refs[1:1 + np_]
    as_refs = refs[1 + np_:1 + 2 * np_]
    (w1x_ref, w1r_ref, w1s_ref, b1_ref, w2_ref, b2_ref, w3_ref, b3_ref,
     lng_ref, lnb_ref, xo_ref) = refs[1 + 2 * np_:]
    x = x_ref[...]
    aggr = ar_refs[0][...]
    aggs = as_refs[0][...]
    for rr in ar_refs[1:]:
        aggr = aggr + rr[...]
    for rr in as_refs[1:]:
        aggs = aggs + rr[...]
    h = (jnp.dot(x, w1x_ref[...], preferred_element_type=jnp.float32)
         + jnp.dot(aggr, w1r_ref[...], preferred_element_type=jnp.float32)
         + jnp.dot(aggs, w1s_ref[...], preferred_element_type=jnp.float32))
    h = _gelu(h + b1_ref[...])
    h = jnp.dot(h, w2_ref[...], preferred_element_type=jnp.float32)
    h = _gelu(h + b2_ref[...])
    h = jnp.dot(h, w3_ref[...], preferred_element_type=jnp.float32) + b3_ref[...]
    xo_ref[...] = x + _ln(h, lng_ref[...], lnb_ref[...])


def _node_mlp(x, aggr_list, aggs_list, w1x, w1r, w1s, b1, w2, b2, w3, b3,
              lng, lnb):
    bn = 2000
    np_ = len(aggr_list)
    row = pl.BlockSpec((bn, H), lambda i: (i, 0))
    wspec = pl.BlockSpec((H, H), lambda i: (0, 0))
    vspec = pl.BlockSpec((1, H), lambda i: (0, 0))
    return pl.pallas_call(
        _node_mlp_body,
        grid=(N // bn,),
        in_specs=([row] + [row] * (2 * np_)
                  + [wspec, wspec, wspec, vspec, wspec, vspec, wspec, vspec,
                     vspec, vspec]),
        out_specs=row,
        out_shape=jax.ShapeDtypeStruct((N, H), jnp.float32),
    )(x, *aggr_list, *aggs_list, w1x, w1r, w1s, b1, w2, b2, w3, b3, lng, lnb)


def kernel(x, edge_attr, edge_index, params):
    p = params
    senders = edge_index[0]
    receivers = edge_index[1]

    eb_w1 = p["eb_W1"]
    a1, ws, wd = eb_w1[:H], eb_w1[H:2 * H], eb_w1[2 * H:]

    xs1, xd1 = _premul(x, ws, wd)

    r1 = lambda a: a.reshape(1, H)
    zeros = jnp.zeros((N, H), jnp.float32)
    eb_args = (a1, r1(p["eb_b1"]), p["eb_W2"], r1(p["eb_b2"]),
               p["eb_W3"], r1(p["eb_b3"]), r1(p["eb_lng"]), r1(p["eb_lnb"]))

    s_p = [senders[i * _EP:(i + 1) * _EP] for i in range(_NP)]
    r_p = [receivers[i * _EP:(i + 1) * _EP] for i in range(_NP)]

    g_p = [_gather_add(xs1, xd1, s_p[i], r_p[i]) for i in range(_NP)]

    eo = None
    en_p = []
    for i in range(_NP):
        eo, en = _edge_mlp(i, edge_attr, g_p[i], *eb_args, eo_prev=eo)
        en_p.append(en)

    aggs = [_scatter(en_p[i], r_p[i], s_p[i], zeros, i == 0)
            for i in range(_NP)]

    nb_w1 = p["nb_W1"]
    half = nb_w1[H:]
    zpad = jnp.zeros((H // 2, H), jnp.float32)
    w1x = nb_w1[:H]
    w1r = jnp.concatenate([half, zpad], axis=0)
    w1s = jnp.concatenate([zpad, half], axis=0)

    x_out = _node_mlp(
        x, [a[0] for a in aggs], [a[1] for a in aggs], w1x, w1r, w1s,
        r1(p["nb_b1"]), p["nb_W2"], r1(p["nb_b2"]), p["nb_W3"],
        r1(p["nb_b3"]), r1(p["nb_lng"]), r1(p["nb_lnb"]))

    return (x_out, eo)


# speedup vs baseline: 6.5187x; 1.0732x over previous
import functools

import jax
import jax.numpy as jnp
from jax import lax
from jax.experimental import pallas as pl
from jax.experimental.pallas import tpu as pltpu
from jax.experimental.pallas import tpu_sc as plsc

N = 10000
E = 320000
H = 128

NC, NS = 2, 16
NW = NC * NS

_EPS = 1e-5

_NP = 2
_EP = E // _NP


def _gelu(h):
    return 0.5 * h * (1.0 + lax.erf(h * 0.7071067811865476))


def _ln(h, g, b):
    m = jnp.mean(h, axis=-1, keepdims=True)
    v = jnp.mean((h - m) ** 2, axis=-1, keepdims=True)
    return (h - m) * lax.rsqrt(v + _EPS) * g + b


def _premul_body(x_ref, ws_ref, wd_ref, xs_ref, xd_ref):
    x = x_ref[...]
    xs_ref[...] = jnp.dot(x, ws_ref[...], preferred_element_type=jnp.float32)
    xd_ref[...] = jnp.dot(x, wd_ref[...], preferred_element_type=jnp.float32)


def _premul(x, ws, wd):
    bn = 2000
    return pl.pallas_call(
        _premul_body,
        grid=(N // bn,),
        in_specs=[
            pl.BlockSpec((bn, H), lambda i: (i, 0)),
            pl.BlockSpec((H, H), lambda i: (0, 0)),
            pl.BlockSpec((H, H), lambda i: (0, 0)),
        ],
        out_specs=[
            pl.BlockSpec((bn, H), lambda i: (i, 0)),
            pl.BlockSpec((bn, H), lambda i: (i, 0)),
        ],
        out_shape=[
            jax.ShapeDtypeStruct((N, H), jnp.float32),
            jax.ShapeDtypeStruct((N, H), jnp.float32),
        ],
    )(x, ws, wd)


_GCH = 128


def _gather_add(xs, xd, s_idx, r_idx):
    mesh = plsc.VectorSubcoreMesh(core_axis_name="c", subcore_axis_name="s")
    ep = s_idx.shape[0]
    epw = ep // NW
    gn = -(-epw // _GCH)

    @functools.partial(
        pl.kernel,
        mesh=mesh,
        out_type=jax.ShapeDtypeStruct((ep, H), jnp.float32),
        scratch_types=[
            pltpu.VMEM((epw,), jnp.int32),
            pltpu.VMEM((epw,), jnp.int32),
            [pltpu.VMEM((_GCH, H), jnp.float32)] * 2,
            [pltpu.VMEM((_GCH, H), jnp.float32)] * 2,
            [pltpu.SemaphoreType.DMA] * 2,
            [pltpu.SemaphoreType.DMA] * 2,
        ],
    )
    def k(xs_hbm, xd_hbm, s_hbm, r_hbm, g_hbm, sall, rall, rows_s, rows_r,
          sem_s, sem_r):
        wid = lax.axis_index("s") * NC + lax.axis_index("c")
        base0 = wid * epw

        pltpu.sync_copy(s_hbm.at[pl.ds(base0, epw)], sall)
        pltpu.sync_copy(r_hbm.at[pl.ds(base0, epw)], rall)

        def off_of(j):
            return lax.min(j * _GCH, epw - _GCH)

        def fire(j, b):
            off = off_of(j)
            pltpu.async_copy(xs_hbm.at[sall.at[pl.ds(off, _GCH)]],
                             rows_s[b], sem_s[b])
            pltpu.async_copy(xd_hbm.at[rall.at[pl.ds(off, _GCH)]],
                             rows_r[b], sem_r[b])

        def proc(j, b):
            pltpu.make_async_copy(xs_hbm.at[sall.at[pl.ds(0, _GCH)]],
                                  rows_s[b], sem_s[b]).wait()
            pltpu.make_async_copy(xd_hbm.at[rall.at[pl.ds(0, _GCH)]],
                                  rows_r[b], sem_r[b]).wait()

            def add_row(rr, c2):
                for cc in range(H // 16):
                    sl = pl.ds(cc * 16, 16)
                    rows_s[b][rr, sl] = rows_s[b][rr, sl] + rows_r[b][rr, sl]
                return c2

            lax.fori_loop(0, _GCH, add_row, 0)
            pltpu.sync_copy(rows_s[b],
                            g_hbm.at[pl.ds(base0 + off_of(j), _GCH), :])

        fire(0, 0)
        npairs = (gn - 1) // 2

        def outer(i, carry):
            for b in range(2):
                j = 2 * i + b
                fire(j + 1, 1 - b)
                proc(j, b)
            return carry

        lax.fori_loop(0, npairs, outer, 0)
        for j in range(2 * npairs, gn):
            if j + 1 < gn:
                fire(j + 1, (j + 1) % 2)
            proc(j, j % 2)

    return k(xs, xd, s_idx, r_idx)


def _edge_mlp_body(ea_ref, g_ref, a1_ref, b1_ref, w2_ref, b2_ref, w3_ref,
                   b3_ref, lng_ref, lnb_ref, eo_ref, en_ref):
    ea = ea_ref[...]
    h = jnp.dot(ea, a1_ref[...], preferred_element_type=jnp.float32)
    h = _gelu(h + g_ref[...] + b1_ref[...])
    h = jnp.dot(h, w2_ref[...], preferred_element_type=jnp.float32)
    h = _gelu(h + b2_ref[...])
    h = jnp.dot(h, w3_ref[...], preferred_element_type=jnp.float32) + b3_ref[...]
    en = _ln(h, lng_ref[...], lnb_ref[...])
    en_ref[...] = en
    eo_ref[...] = ea + en


def _edge_mlp_body_aliased(ea_ref, g_ref, a1_ref, b1_ref, w2_ref, b2_ref,
                           w3_ref, b3_ref, lng_ref, lnb_ref, eoprev_ref,
                           eo_ref, en_ref):
    _edge_mlp_body(ea_ref, g_ref, a1_ref, b1_ref, w2_ref, b2_ref, w3_ref,
                   b3_ref, lng_ref, lnb_ref, eo_ref, en_ref)


def _edge_mlp(piece, ea, g, a1, b1, w2, b2, w3, b3, lng, lnb, eo_prev):
    be = 1600
    ep = g.shape[0]
    off = piece * ep // be
    rowg = pl.BlockSpec((be, H), lambda i: (i, 0))
    rowe = pl.BlockSpec((be, H), lambda i: (i + off, 0))
    wspec = pl.BlockSpec((H, H), lambda i: (0, 0))
    vspec = pl.BlockSpec((1, H), lambda i: (0, 0))
    in_specs = [rowe, rowg, wspec, vspec, wspec, vspec, wspec, vspec, vspec,
                vspec]
    args = [ea, g, a1, b1, w2, b2, w3, b3, lng, lnb]
    if eo_prev is None:
        body = _edge_mlp_body
        aliases = {}
    else:
        body = _edge_mlp_body_aliased
        in_specs = in_specs + [pl.BlockSpec((8, H), lambda i: (0, 0))]
        args = args + [eo_prev]
        aliases = {10: 0}
    return pl.pallas_call(
        body,
        grid=(ep // be,),
        in_specs=in_specs,
        out_specs=[rowe, rowg],
        out_shape=[
            jax.ShapeDtypeStruct((E, H), jnp.float32),
            jax.ShapeDtypeStruct((ep, H), jnp.float32),
        ],
        input_output_aliases=aliases,
    )(*args)


_SCH = 80


def _scatter(en, ridx, sidx, zeros, first=False):
    mesh = plsc.VectorSubcoreMesh(core_axis_name="c", subcore_axis_name="s")
    ep = en.shape[0]
    ept = ep // NS
    sn = ept // _SCH

    @functools.partial(
        pl.kernel,
        mesh=mesh,
        out_type=jax.ShapeDtypeStruct((2, N, H), jnp.float32),
        scratch_types=[
            [pltpu.VMEM((_SCH,), jnp.int32)] * 2,
            [pltpu.VMEM((_SCH, H), jnp.float32)] * 2,
            pltpu.VMEM_SHARED((N, H), jnp.float32),
            [pltpu.SemaphoreType.DMA] * 2,
            [pltpu.SemaphoreType.DMA] * 2,
        ],
    )
    def k(en_hbm, r_hbm, s_hbm, z_hbm, out_hbm, idxbuf, rows, agg, sem_i,
          sem_d):
        c = lax.axis_index("c")
        t = lax.axis_index("s")

        @pl.when(t == 0)
        def _():
            pltpu.sync_copy(z_hbm, agg)

        plsc.subcore_barrier()

        def fire(j, b):
            base = t * ept + j * _SCH

            @pl.when(c == 0)
            def _():
                pltpu.async_copy(r_hbm.at[pl.ds(base, _SCH)], idxbuf[b],
                                 sem_i[b])

            @pl.when(c == 1)
            def _():
                pltpu.async_copy(s_hbm.at[pl.ds(base, _SCH)], idxbuf[b],
                                 sem_i[b])

            pltpu.async_copy(en_hbm.at[pl.ds(base, _SCH), :], rows[b],
                             sem_d[b])

        def scat(b):
            pltpu.make_async_copy(r_hbm.at[pl.ds(0, _SCH)], idxbuf[b],
                                  sem_i[b]).wait()
            pltpu.make_async_copy(en_hbm.at[pl.ds(0, _SCH), :], rows[b],
                                  sem_d[b]).wait()
            pltpu.sync_copy(rows[b], agg.at[idxbuf[b]], add=True)

        fire(0, 0)
        npairs = (sn - 1) // 2

        def outer(i, carry):
            for b in range(2):
                j = 2 * i + b
                fire(j + 1, 1 - b)
                scat(b)
            return carry

        lax.fori_loop(0, npairs, outer, 0)
        for j in range(2 * npairs, sn):
            if j + 1 < sn:
                fire(j + 1, (j + 1) % 2)
            scat(j % 2)
        plsc.subcore_barrier()

        @pl.when(t == 0)
        def _():
            pltpu.sync_copy(agg, out_hbm.at[c])

    return k(en, ridx, sidx, zeros)


def _node_mlp_body(*refs):
    x_ref = refs[0]
    np_ = _NP
    ar_refs = refs[1:1 + np_]
    as_refs = refs[1 + np_:1 + 2 * np_]
    (w1x_ref, w1r_ref, w1s_ref, b1_ref, w2_ref, b2_ref, w3_ref, b3_ref,
     lng_ref, lnb_ref, xo_ref) = refs[1 + 2 * np_:]
    x = x_ref[...]
    aggr = ar_refs[0][...]
    aggs = as_refs[0][...]
    for rr in ar_refs[1:]:
        aggr = aggr + rr[...]
    for rr in as_refs[1:]:
        aggs = aggs + rr[...]
    h = (jnp.dot(x, w1x_ref[...], preferred_element_type=jnp.float32)
         + jnp.dot(aggr, w1r_ref[...], preferred_element_type=jnp.float32)
         + jnp.dot(aggs, w1s_ref[...], preferred_element_type=jnp.float32))
    h = _gelu(h + b1_ref[...])
    h = jnp.dot(h, w2_ref[...], preferred_element_type=jnp.float32)
    h = _gelu(h + b2_ref[...])
    h = jnp.dot(h, w3_ref[...], preferred_element_type=jnp.float32) + b3_ref[...]
    xo_ref[...] = x + _ln(h, lng_ref[...], lnb_ref[...])


def _node_mlp(x, aggr_list, aggs_list, w1x, w1r, w1s, b1, w2, b2, w3, b3,
              lng, lnb):
    bn = 2000
    np_ = len(aggr_list)
    row = pl.BlockSpec((bn, H), lambda i: (i, 0))
    wspec = pl.BlockSpec((H, H), lambda i: (0, 0))
    vspec = pl.BlockSpec((1, H), lambda i: (0, 0))
    return pl.pallas_call(
        _node_mlp_body,
        grid=(N // bn,),
        in_specs=([row] + [row] * (2 * np_)
                  + [wspec, wspec, wspec, vspec, wspec, vspec, wspec, vspec,
                     vspec, vspec]),
        out_specs=row,
        out_shape=jax.ShapeDtypeStruct((N, H), jnp.float32),
    )(x, *aggr_list, *aggs_list, w1x, w1r, w1s, b1, w2, b2, w3, b3, lng, lnb)


def kernel(x, edge_attr, edge_index, params):
    p = params
    senders = edge_index[0]
    receivers = edge_index[1]

    eb_w1 = p["eb_W1"]
    a1, ws, wd = eb_w1[:H], eb_w1[H:2 * H], eb_w1[2 * H:]

    xs1, xd1 = _premul(x, ws, wd)

    r1 = lambda a: a.reshape(1, H)
    zeros = jnp.zeros((N, H), jnp.float32)
    eb_args = (a1, r1(p["eb_b1"]), p["eb_W2"], r1(p["eb_b2"]),
               p["eb_W3"], r1(p["eb_b3"]), r1(p["eb_lng"]), r1(p["eb_lnb"]))

    s_p = [senders[i * _EP:(i + 1) * _EP] for i in range(_NP)]
    r_p = [receivers[i * _EP:(i + 1) * _EP] for i in range(_NP)]

    g_p = [_gather_add(xs1, xd1, s_p[i], r_p[i]) for i in range(_NP)]

    eo = None
    en_p = []
    for i in range(_NP):
        eo, en = _edge_mlp(i, edge_attr, g_p[i], *eb_args, eo_prev=eo)
        en_p.append(en)

    aggs = [_scatter(en_p[i], r_p[i], s_p[i], zeros, i == 0)
            for i in range(_NP)]

    nb_w1 = p["nb_W1"]
    half = nb_w1[H:]
    zpad = jnp.zeros((H // 2, H), jnp.float32)
    w1x = nb_w1[:H]
    w1r = jnp.concatenate([half, zpad], axis=0)
    w1s = jnp.concatenate([zpad, half], axis=0)

    x_out = _node_mlp(
        x, [a[0] for a in aggs], [a[1] for a in aggs], w1x, w1r, w1s,
        r1(p["nb_b1"]), p["nb_W2"], r1(p["nb_b2"]), p["nb_W3"],
        r1(p["nb_b3"]), r1(p["nb_lng"]), r1(p["nb_lnb"]))

    return (x_out, eo)


# speedup vs baseline: 6.5739x; 1.0085x over previous
import functools

import jax
import jax.numpy as jnp
from jax import lax
from jax.experimental import pallas as pl
from jax.experimental.pallas import tpu as pltpu
from jax.experimental.pallas import tpu_sc as plsc

N = 10000
E = 320000
H = 128

NC, NS = 2, 16
NW = NC * NS

_EPS = 1e-5

_NP = 2
_EP = E // _NP


def _gelu(h):
    return 0.5 * h * (1.0 + lax.erf(h * 0.7071067811865476))


def _ln(h, g, b):
    m = jnp.mean(h, axis=-1, keepdims=True)
    v = jnp.mean((h - m) ** 2, axis=-1, keepdims=True)
    return (h - m) * lax.rsqrt(v + _EPS) * g + b


def _premul_body(x_ref, ws_ref, wd_ref, xs_ref, xd_ref):
    x = x_ref[...]
    xs_ref[...] = jnp.dot(x, ws_ref[...], preferred_element_type=jnp.float32)
    xd_ref[...] = jnp.dot(x, wd_ref[...], preferred_element_type=jnp.float32)


def _premul(x, ws, wd):
    bn = 2000
    return pl.pallas_call(
        _premul_body,
        grid=(N // bn,),
        in_specs=[
            pl.BlockSpec((bn, H), lambda i: (i, 0)),
            pl.BlockSpec((H, H), lambda i: (0, 0)),
            pl.BlockSpec((H, H), lambda i: (0, 0)),
        ],
        out_specs=[
            pl.BlockSpec((bn, H), lambda i: (i, 0)),
            pl.BlockSpec((bn, H), lambda i: (i, 0)),
        ],
        out_shape=[
            jax.ShapeDtypeStruct((N, H), jnp.float32),
            jax.ShapeDtypeStruct((N, H), jnp.float32),
        ],
    )(x, ws, wd)


_GCH = 128


def _gather_add(xs, xd, s_idx, r_idx):
    mesh = plsc.VectorSubcoreMesh(core_axis_name="c", subcore_axis_name="s")
    ep = s_idx.shape[0]
    epw = ep // NW
    gn = -(-epw // _GCH)

    @functools.partial(
        pl.kernel,
        mesh=mesh,
        out_type=jax.ShapeDtypeStruct((ep, H), jnp.float32),
        scratch_types=[
            pltpu.VMEM((epw,), jnp.int32),
            pltpu.VMEM((epw,), jnp.int32),
            [pltpu.VMEM((_GCH, H), jnp.float32)] * 2,
            [pltpu.VMEM((_GCH, H), jnp.float32)] * 2,
            [pltpu.SemaphoreType.DMA] * 2,
            [pltpu.SemaphoreType.DMA] * 2,
        ],
    )
    def k(xs_hbm, xd_hbm, s_hbm, r_hbm, g_hbm, sall, rall, rows_s, rows_r,
          sem_s, sem_r):
        wid = lax.axis_index("s") * NC + lax.axis_index("c")
        base0 = wid * epw

        pltpu.sync_copy(s_hbm.at[pl.ds(base0, epw)], sall)
        pltpu.sync_copy(r_hbm.at[pl.ds(base0, epw)], rall)

        def off_of(j):
            return lax.min(j * _GCH, epw - _GCH)

        def fire(j, b):
            off = off_of(j)
            pltpu.async_copy(xs_hbm.at[sall.at[pl.ds(off, _GCH)]],
                             rows_s[b], sem_s[b])
            pltpu.async_copy(xd_hbm.at[rall.at[pl.ds(off, _GCH)]],
                             rows_r[b], sem_r[b])

        def proc(j, b):
            pltpu.make_async_copy(xs_hbm.at[sall.at[pl.ds(0, _GCH)]],
                                  rows_s[b], sem_s[b]).wait()
            pltpu.make_async_copy(xd_hbm.at[rall.at[pl.ds(0, _GCH)]],
                                  rows_r[b], sem_r[b]).wait()

            def add_row(rr, c2):
                for cc in range(H // 16):
                    sl = pl.ds(cc * 16, 16)
                    rows_s[b][rr, sl] = rows_s[b][rr, sl] + rows_r[b][rr, sl]
                return c2

            lax.fori_loop(0, _GCH, add_row, 0)
            pltpu.sync_copy(rows_s[b],
                            g_hbm.at[pl.ds(base0 + off_of(j), _GCH), :])

        fire(0, 0)
        npairs = (gn - 1) // 2

        def outer(i, carry):
            for b in range(2):
                j = 2 * i + b
                fire(j + 1, 1 - b)
                proc(j, b)
            return carry

        lax.fori_loop(0, npairs, outer, 0)
        for j in range(2 * npairs, gn):
            if j + 1 < gn:
                fire(j + 1, (j + 1) % 2)
            proc(j, j % 2)

    return k(xs, xd, s_idx, r_idx)


def _edge_mlp_body(ea_ref, g_ref, a1_ref, b1_ref, w2_ref, b2_ref, w3_ref,
                   b3_ref, lng_ref, lnb_ref, eo_ref, en_ref):
    ea = ea_ref[...]
    h = jnp.dot(ea, a1_ref[...], preferred_element_type=jnp.float32)
    h = _gelu(h + g_ref[...] + b1_ref[...])
    h = jnp.dot(h, w2_ref[...], preferred_element_type=jnp.float32)
    h = _gelu(h + b2_ref[...])
    h = jnp.dot(h, w3_ref[...], preferred_element_type=jnp.float32) + b3_ref[...]
    en = _ln(h, lng_ref[...], lnb_ref[...])
    en_ref[...] = en
    eo_ref[...] = ea + en


def _edge_mlp_body_aliased(ea_ref, g_ref, a1_ref, b1_ref, w2_ref, b2_ref,
                           w3_ref, b3_ref, lng_ref, lnb_ref, eoprev_ref,
                           eo_ref, en_ref):
    _edge_mlp_body(ea_ref, g_ref, a1_ref, b1_ref, w2_ref, b2_ref, w3_ref,
                   b3_ref, lng_ref, lnb_ref, eo_ref, en_ref)


def _edge_mlp(piece, ea, g, a1, b1, w2, b2, w3, b3, lng, lnb, eo_prev):
    be = 1600
    ep = g.shape[0]
    off = piece * ep // be
    rowg = pl.BlockSpec((be, H), lambda i: (i, 0))
    rowe = pl.BlockSpec((be, H), lambda i: (i + off, 0))
    wspec = pl.BlockSpec((H, H), lambda i: (0, 0))
    vspec = pl.BlockSpec((1, H), lambda i: (0, 0))
    in_specs = [rowe, rowg, wspec, vspec, wspec, vspec, wspec, vspec, vspec,
                vspec]
    args = [ea, g, a1, b1, w2, b2, w3, b3, lng, lnb]
    if eo_prev is None:
        body = _edge_mlp_body
        aliases = {}
    else:
        body = _edge_mlp_body_aliased
        in_specs = in_specs + [pl.BlockSpec((8, H), lambda i: (0, 0))]
        args = args + [eo_prev]
        aliases = {10: 0}
    return pl.pallas_call(
        body,
        grid=(ep // be,),
        in_specs=in_specs,
        out_specs=[rowe, rowg],
        out_shape=[
            jax.ShapeDtypeStruct((E, H), jnp.float32),
            jax.ShapeDtypeStruct((ep, H), jnp.float32),
        ],
        input_output_aliases=aliases,
    )(*args)


_SCH = 80


_ZR = 125


def _scatter(en, ridx, sidx):
    mesh = plsc.VectorSubcoreMesh(core_axis_name="c", subcore_axis_name="s")
    ep = en.shape[0]
    ept = ep // NS
    sn = ept // _SCH

    @functools.partial(
        pl.kernel,
        mesh=mesh,
        out_type=jax.ShapeDtypeStruct((2, N, H), jnp.float32),
        scratch_types=[
            [pltpu.VMEM((_SCH,), jnp.int32)] * 2,
            [pltpu.VMEM((_SCH, H), jnp.float32)] * 2,
            pltpu.VMEM((_ZR, H), jnp.float32),
            pltpu.VMEM_SHARED((N, H), jnp.float32),
            [pltpu.SemaphoreType.DMA] * 2,
            [pltpu.SemaphoreType.DMA] * 2,
        ],
    )
    def k(en_hbm, r_hbm, s_hbm, out_hbm, idxbuf, rows, zbuf, agg, sem_i,
          sem_d):
        c = lax.axis_index("c")
        t = lax.axis_index("s")

        def zrow(rr, carry):
            for cc in range(H // 16):
                zbuf[rr, pl.ds(cc * 16, 16)] = jnp.zeros((16,), jnp.float32)
            return carry

        lax.fori_loop(0, _ZR, zrow, 0)
        nz = N // NS // _ZR
        for zz in range(nz):
            pltpu.sync_copy(
                zbuf, agg.at[pl.ds((t * nz + zz) * _ZR, _ZR), :])

        plsc.subcore_barrier()

        def fire(j, b):
            base = t * ept + j * _SCH

            @pl.when(c == 0)
            def _():
                pltpu.async_copy(r_hbm.at[pl.ds(base, _SCH)], idxbuf[b],
                                 sem_i[b])

            @pl.when(c == 1)
            def _():
                pltpu.async_copy(s_hbm.at[pl.ds(base, _SCH)], idxbuf[b],
                                 sem_i[b])

            pltpu.async_copy(en_hbm.at[pl.ds(base, _SCH), :], rows[b],
                             sem_d[b])

        def scat(b):
            pltpu.make_async_copy(r_hbm.at[pl.ds(0, _SCH)], idxbuf[b],
                                  sem_i[b]).wait()
            pltpu.make_async_copy(en_hbm.at[pl.ds(0, _SCH), :], rows[b],
                                  sem_d[b]).wait()
            pltpu.sync_copy(rows[b], agg.at[idxbuf[b]], add=True)

        fire(0, 0)
        npairs = (sn - 1) // 2

        def outer(i, carry):
            for b in range(2):
                j = 2 * i + b
                fire(j + 1, 1 - b)
                scat(b)
            return carry

        lax.fori_loop(0, npairs, outer, 0)
        for j in range(2 * npairs, sn):
            if j + 1 < sn:
                fire(j + 1, (j + 1) % 2)
            scat(j % 2)
        plsc.subcore_barrier()

        @pl.when(t == 0)
        def _():
            pltpu.sync_copy(agg, out_hbm.at[c])

    return k(en, ridx, sidx)


def _node_mlp_body(*refs):
    x_ref = refs[0]
    np_ = _NP
    ar_refs = refs[1:1 + np_]
    as_refs = refs[1 + np_:1 + 2 * np_]
    (w1x_ref, w1r_ref, w1s_ref, b1_ref, w2_ref, b2_ref, w3_ref, b3_ref,
     lng_ref, lnb_ref, xo_ref) = refs[1 + 2 * np_:]
    x = x_ref[...]
    aggr = ar_refs[0][...]
    aggs = as_refs[0][...]
    for rr in ar_refs[1:]:
        aggr = aggr + rr[...]
    for rr in as_refs[1:]:
        aggs = aggs + rr[...]
    h = (jnp.dot(x, w1x_ref[...], preferred_element_type=jnp.float32)
         + jnp.dot(aggr, w1r_ref[...], preferred_element_type=jnp.float32)
         + jnp.dot(aggs, w1s_ref[...], preferred_element_type=jnp.float32))
    h = _gelu(h + b1_ref[...])
    h = jnp.dot(h, w2_ref[...], preferred_element_type=jnp.float32)
    h = _gelu(h + b2_ref[...])
    h = jnp.dot(h, w3_ref[...], preferred_element_type=jnp.float32) + b3_ref[...]
    xo_ref[...] = x + _ln(h, lng_ref[...], lnb_ref[...])


def _node_mlp(x, aggr_list, aggs_list, w1x, w1r, w1s, b1, w2, b2, w3, b3,
              lng, lnb):
    bn = 2000
    np_ = len(aggr_list)
    row = pl.BlockSpec((bn, H), lambda i: (i, 0))
    wspec = pl.BlockSpec((H, H), lambda i: (0, 0))
    vspec = pl.BlockSpec((1, H), lambda i: (0, 0))
    return pl.pallas_call(
        _node_mlp_body,
        grid=(N // bn,),
        in_specs=([row] + [row] * (2 * np_)
                  + [wspec, wspec, wspec, vspec, wspec, vspec, wspec, vspec,
                     vspec, vspec]),
        out_specs=row,
        out_shape=jax.ShapeDtypeStruct((N, H), jnp.float32),
    )(x, *aggr_list, *aggs_list, w1x, w1r, w1s, b1, w2, b2, w3, b3, lng, lnb)


def kernel(x, edge_attr, edge_index, params):
    p = params
    senders = edge_index[0]
    receivers = edge_index[1]

    eb_w1 = p["eb_W1"]
    a1, ws, wd = eb_w1[:H], eb_w1[H:2 * H], eb_w1[2 * H:]

    xs1, xd1 = _premul(x, ws, wd)

    r1 = lambda a: a.reshape(1, H)
    eb_args = (a1, r1(p["eb_b1"]), p["eb_W2"], r1(p["eb_b2"]),
               p["eb_W3"], r1(p["eb_b3"]), r1(p["eb_lng"]), r1(p["eb_lnb"]))

    s_p = [senders[i * _EP:(i + 1) * _EP] for i in range(_NP)]
    r_p = [receivers[i * _EP:(i + 1) * _EP] for i in range(_NP)]

    g_p = [_gather_add(xs1, xd1, s_p[i], r_p[i]) for i in range(_NP)]

    eo = None
    en_p = []
    for i in range(_NP):
        eo, en = _edge_mlp(i, edge_attr, g_p[i], *eb_args, eo_prev=eo)
        en_p.append(en)

    aggs = [_scatter(en_p[i], r_p[i], s_p[i]) for i in range(_NP)]

    nb_w1 = p["nb_W1"]
    half = nb_w1[H:]
    zpad = jnp.zeros((H // 2, H), jnp.float32)
    w1x = nb_w1[:H]
    w1r = jnp.concatenate([half, zpad], axis=0)
    w1s = jnp.concatenate([zpad, half], axis=0)

    x_out = _node_mlp(
        x, [a[0] for a in aggs], [a[1] for a in aggs], w1x, w1r, w1s,
        r1(p["nb_b1"]), p["nb_W2"], r1(p["nb_b2"]), p["nb_W3"],
        r1(p["nb_b3"]), r1(p["nb_lng"]), r1(p["nb_lnb"]))

    return (x_out, eo)


# speedup vs baseline: 6.5769x; 1.0004x over previous
import functools

import jax
import jax.numpy as jnp
from jax import lax
from jax.experimental import pallas as pl
from jax.experimental.pallas import tpu as pltpu
from jax.experimental.pallas import tpu_sc as plsc

N = 10000
E = 320000
H = 128

NC, NS = 2, 16
NW = NC * NS

_EPS = 1e-5

_NP = 2
_EP = E // _NP


def _gelu(h):
    return 0.5 * h * (1.0 + lax.erf(h * 0.7071067811865476))


def _ln(h, g, b):
    m = jnp.mean(h, axis=-1, keepdims=True)
    v = jnp.mean((h - m) ** 2, axis=-1, keepdims=True)
    return (h - m) * lax.rsqrt(v + _EPS) * g + b


def _premul_body(x_ref, ws_ref, wd_ref, xs_ref, xd_ref):
    x = x_ref[...]
    xs_ref[...] = jnp.dot(x, ws_ref[...], preferred_element_type=jnp.float32)
    xd_ref[...] = jnp.dot(x, wd_ref[...], preferred_element_type=jnp.float32)


def _premul(x, ws, wd):
    bn = 2000
    return pl.pallas_call(
        _premul_body,
        grid=(N // bn,),
        in_specs=[
            pl.BlockSpec((bn, H), lambda i: (i, 0)),
            pl.BlockSpec((H, H), lambda i: (0, 0)),
            pl.BlockSpec((H, H), lambda i: (0, 0)),
        ],
        out_specs=[
            pl.BlockSpec((bn, H), lambda i: (i, 0)),
            pl.BlockSpec((bn, H), lambda i: (i, 0)),
        ],
        out_shape=[
            jax.ShapeDtypeStruct((N, H), jnp.float32),
            jax.ShapeDtypeStruct((N, H), jnp.float32),
        ],
    )(x, ws, wd)


_GCH = 128


def _gather_add(xs, xd, s_idx, r_idx):
    mesh = plsc.VectorSubcoreMesh(core_axis_name="c", subcore_axis_name="s")
    ep = s_idx.shape[0]
    epw = ep // NW
    gn = -(-epw // _GCH)

    @functools.partial(
        pl.kernel,
        mesh=mesh,
        out_type=jax.ShapeDtypeStruct((ep, H), jnp.float32),
        scratch_types=[
            pltpu.VMEM((epw,), jnp.int32),
            pltpu.VMEM((epw,), jnp.int32),
            [pltpu.VMEM((_GCH, H), jnp.float32)] * 2,
            [pltpu.VMEM((_GCH, H), jnp.float32)] * 2,
            [pltpu.SemaphoreType.DMA] * 2,
            [pltpu.SemaphoreType.DMA] * 2,
            [pltpu.SemaphoreType.DMA] * 2,
        ],
    )
    def k(xs_hbm, xd_hbm, s_hbm, r_hbm, g_hbm, sall, rall, rows_s, rows_r,
          sem_s, sem_r, sem_st):
        wid = lax.axis_index("s") * NC + lax.axis_index("c")
        base0 = wid * epw

        pltpu.sync_copy(s_hbm.at[pl.ds(base0, epw)], sall)
        pltpu.sync_copy(r_hbm.at[pl.ds(base0, epw)], rall)

        def off_of(j):
            return lax.min(j * _GCH, epw - _GCH)

        def fire(j, b):
            off = off_of(j)
            pltpu.async_copy(xs_hbm.at[sall.at[pl.ds(off, _GCH)]],
                             rows_s[b], sem_s[b])
            pltpu.async_copy(xd_hbm.at[rall.at[pl.ds(off, _GCH)]],
                             rows_r[b], sem_r[b])

        def proc(j, b):
            pltpu.make_async_copy(xs_hbm.at[sall.at[pl.ds(0, _GCH)]],
                                  rows_s[b], sem_s[b]).wait()
            pltpu.make_async_copy(xd_hbm.at[rall.at[pl.ds(0, _GCH)]],
                                  rows_r[b], sem_r[b]).wait()

            def add_row(rr, c2):
                for cc in range(H // 16):
                    sl = pl.ds(cc * 16, 16)
                    rows_s[b][rr, sl] = rows_s[b][rr, sl] + rows_r[b][rr, sl]
                return c2

            lax.fori_loop(0, _GCH, add_row, 0)
            pltpu.async_copy(rows_s[b],
                             g_hbm.at[pl.ds(base0 + off_of(j), _GCH), :],
                             sem_st[b])

        def drain_store(b):
            pltpu.make_async_copy(rows_s[b],
                                  g_hbm.at[pl.ds(base0, _GCH), :],
                                  sem_st[b]).wait()

        fire(0, 0)
        npairs = (gn - 1) // 2

        def outer(i, carry):
            for b in range(2):
                j = 2 * i + b
                if b == 0:
                    @pl.when(i > 0)
                    def _():
                        drain_store(1)
                else:
                    drain_store(0)
                fire(j + 1, 1 - b)
                proc(j, b)
            return carry

        lax.fori_loop(0, npairs, outer, 0)
        for j in range(2 * npairs, gn):
            drain_store((j + 1) % 2)
            if j + 1 < gn:
                fire(j + 1, (j + 1) % 2)
            proc(j, j % 2)
        drain_store((gn - 1) % 2)

    return k(xs, xd, s_idx, r_idx)


def _edge_mlp_body(ea_ref, g_ref, a1_ref, b1_ref, w2_ref, b2_ref, w3_ref,
                   b3_ref, lng_ref, lnb_ref, eo_ref, en_ref):
    ea = ea_ref[...]
    h = jnp.dot(ea, a1_ref[...], preferred_element_type=jnp.float32)
    h = _gelu(h + g_ref[...] + b1_ref[...])
    h = jnp.dot(h, w2_ref[...], preferred_element_type=jnp.float32)
    h = _gelu(h + b2_ref[...])
    h = jnp.dot(h, w3_ref[...], preferred_element_type=jnp.float32) + b3_ref[...]
    en = _ln(h, lng_ref[...], lnb_ref[...])
    en_ref[...] = en
    eo_ref[...] = ea + en


def _edge_mlp_body_aliased(ea_ref, g_ref, a1_ref, b1_ref, w2_ref, b2_ref,
                           w3_ref, b3_ref, lng_ref, lnb_ref, eoprev_ref,
                           eo_ref, en_ref):
    _edge_mlp_body(ea_ref, g_ref, a1_ref, b1_ref, w2_ref, b2_ref, w3_ref,
                   b3_ref, lng_ref, lnb_ref, eo_ref, en_ref)


def _edge_mlp(piece, ea, g, a1, b1, w2, b2, w3, b3, lng, lnb, eo_prev):
    be = 1600
    ep = g.shape[0]
    off = piece * ep // be
    rowg = pl.BlockSpec((be, H), lambda i: (i, 0))
    rowe = pl.BlockSpec((be, H), lambda i: (i + off, 0))
    wspec = pl.BlockSpec((H, H), lambda i: (0, 0))
    vspec = pl.BlockSpec((1, H), lambda i: (0, 0))
    in_specs = [rowe, rowg, wspec, vspec, wspec, vspec, wspec, vspec, vspec,
                vspec]
    args = [ea, g, a1, b1, w2, b2, w3, b3, lng, lnb]
    if eo_prev is None:
        body = _edge_mlp_body
        aliases = {}
    else:
        body = _edge_mlp_body_aliased
        in_specs = in_specs + [pl.BlockSpec((8, H), lambda i: (0, 0))]
        args = args + [eo_prev]
        aliases = {10: 0}
    return pl.pallas_call(
        body,
        grid=(ep // be,),
        in_specs=in_specs,
        out_specs=[rowe, rowg],
        out_shape=[
            jax.ShapeDtypeStruct((E, H), jnp.float32),
            jax.ShapeDtypeStruct((ep, H), jnp.float32),
        ],
        input_output_aliases=aliases,
    )(*args)


_SCH = 80


_ZR = 125


def _scatter(en, ridx, sidx):
    mesh = plsc.VectorSubcoreMesh(core_axis_name="c", subcore_axis_name="s")
    ep = en.shape[0]
    ept = ep // NS
    sn = ept // _SCH

    @functools.partial(
        pl.kernel,
        mesh=mesh,
        out_type=jax.ShapeDtypeStruct((2, N, H), jnp.float32),
        scratch_types=[
            [pltpu.VMEM((_SCH,), jnp.int32)] * 2,
            [pltpu.VMEM((_SCH, H), jnp.float32)] * 2,
            pltpu.VMEM((_ZR, H), jnp.float32),
            pltpu.VMEM_SHARED((N, H), jnp.float32),
            [pltpu.SemaphoreType.DMA] * 2,
            [pltpu.SemaphoreType.DMA] * 2,
            [pltpu.SemaphoreType.DMA] * 2,
        ],
    )
    def k(en_hbm, r_hbm, s_hbm, out_hbm, idxbuf, rows, zbuf, agg, sem_i,
          sem_d, sem_sc):
        c = lax.axis_index("c")
        t = lax.axis_index("s")

        def zrow(rr, carry):
            for cc in range(H // 16):
                zbuf[rr, pl.ds(cc * 16, 16)] = jnp.zeros((16,), jnp.float32)
            return carry

        lax.fori_loop(0, _ZR, zrow, 0)
        nz = N // NS // _ZR
        for zz in range(nz):
            pltpu.sync_copy(
                zbuf, agg.at[pl.ds((t * nz + zz) * _ZR, _ZR), :])

        plsc.subcore_barrier()

        def fire(j, b):
            base = t * ept + j * _SCH

            @pl.when(c == 0)
            def _():
                pltpu.async_copy(r_hbm.at[pl.ds(base, _SCH)], idxbuf[b],
                                 sem_i[b])

            @pl.when(c == 1)
            def _():
                pltpu.async_copy(s_hbm.at[pl.ds(base, _SCH)], idxbuf[b],
                                 sem_i[b])

            pltpu.async_copy(en_hbm.at[pl.ds(base, _SCH), :], rows[b],
                             sem_d[b])

        def scat(b):
            pltpu.make_async_copy(r_hbm.at[pl.ds(0, _SCH)], idxbuf[b],
                                  sem_i[b]).wait()
            pltpu.make_async_copy(en_hbm.at[pl.ds(0, _SCH), :], rows[b],
                                  sem_d[b]).wait()
            pltpu.async_copy(rows[b], agg.at[idxbuf[b]], sem_sc[b], add=True)

        def drain_scat(b):
            pltpu.make_async_copy(rows[b], agg.at[idxbuf[b]],
                                  sem_sc[b]).wait()

        fire(0, 0)
        npairs = (sn - 1) // 2

        def outer(i, carry):
            for b in range(2):
                j = 2 * i + b
                if b == 0:
                    @pl.when(i > 0)
                    def _():
                        drain_scat(1)
                else:
                    drain_scat(0)
                fire(j + 1, 1 - b)
                scat(b)
            return carry

        lax.fori_loop(0, npairs, outer, 0)
        for j in range(2 * npairs, sn):
            drain_scat((j + 1) % 2)
            if j + 1 < sn:
                fire(j + 1, (j + 1) % 2)
            scat(j % 2)
        drain_scat((sn - 1) % 2)
        plsc.subcore_barrier()

        @pl.when(t == 0)
        def _():
            pltpu.sync_copy(agg, out_hbm.at[c])

    return k(en, ridx, sidx)


def _node_mlp_body(*refs):
    x_ref = refs[0]
    np_ = _NP
    ar_refs = refs[1:1 + np_]
    as_refs = refs[1 + np_:1 + 2 * np_]
    (w1x_ref, w1r_ref, w1s_ref, b1_ref, w2_ref, b2_ref, w3_ref, b3_ref,
     lng_ref, lnb_ref, xo_ref) = refs[1 + 2 * np_:]
    x = x_ref[...]
    aggr = ar_refs[0][...]
    aggs = as_refs[0][...]
    for rr in ar_refs[1:]:
        aggr = aggr + rr[...]
    for rr in as_refs[1:]:
        aggs = aggs + rr[...]
    h = (jnp.dot(x, w1x_ref[...], preferred_element_type=jnp.float32)
         + jnp.dot(aggr, w1r_ref[...], preferred_element_type=jnp.float32)
         + jnp.dot(aggs, w1s_ref[...], preferred_element_type=jnp.float32))
    h = _gelu(h + b1_ref[...])
    h = jnp.dot(h, w2_ref[...], preferred_element_type=jnp.float32)
    h = _gelu(h + b2_ref[...])
    h = jnp.dot(h, w3_ref[...], preferred_element_type=jnp.float32) + b3_ref[...]
    xo_ref[...] = x + _ln(h, lng_ref[...], lnb_ref[...])


def _node_mlp(x, aggr_list, aggs_list, w1x, w1r, w1s, b1, w2, b2, w3, b3,
              lng, lnb):
    bn = 2000
    np_ = len(aggr_list)
    row = pl.BlockSpec((bn, H), lambda i: (i, 0))
    wspec = pl.BlockSpec((H, H), lambda i: (0, 0))
    vspec = pl.BlockSpec((1, H), lambda i: (0, 0))
    return pl.pallas_call(
        _node_mlp_body,
        grid=(N // bn,),
        in_specs=([row] + [row] * (2 * np_)
                  + [wspec, wspec, wspec, vspec, wspec, vspec, wspec, vspec,
                     vspec, vspec]),
        out_specs=row,
        out_shape=jax.ShapeDtypeStruct((N, H), jnp.float32),
    )(x, *aggr_list, *aggs_list, w1x, w1r, w1s, b1, w2, b2, w3, b3, lng, lnb)


def kernel(x, edge_attr, edge_index, params):
    p = params
    senders = edge_index[0]
    receivers = edge_index[1]

    eb_w1 = p["eb_W1"]
    a1, ws, wd = eb_w1[:H], eb_w1[H:2 * H], eb_w1[2 * H:]

    xs1, xd1 = _premul(x, ws, wd)

    r1 = lambda a: a.reshape(1, H)
    eb_args = (a1, r1(p["eb_b1"]), p["eb_W2"], r1(p["eb_b2"]),
               p["eb_W3"], r1(p["eb_b3"]), r1(p["eb_lng"]), r1(p["eb_lnb"]))

    s_p = [senders[i * _EP:(i + 1) * _EP] for i in range(_NP)]
    r_p = [receivers[i * _EP:(i + 1) * _EP] for i in range(_NP)]

    g_p = [_gather_add(xs1, xd1, s_p[i], r_p[i]) for i in range(_NP)]

    eo = None
    en_p = []
    for i in range(_NP):
        eo, en = _edge_mlp(i, edge_attr, g_p[i], *eb_args, eo_prev=eo)
        en_p.append(en)

    aggs = [_scatter(en_p[i], r_p[i], s_p[i]) for i in range(_NP)]

    nb_w1 = p["nb_W1"]
    half = nb_w1[H:]
    zpad = jnp.zeros((H // 2, H), jnp.float32)
    w1x = nb_w1[:H]
    w1r = jnp.concatenate([half, zpad], axis=0)
    w1s = jnp.concatenate([zpad, half], axis=0)

    x_out = _node_mlp(
        x, [a[0] for a in aggs], [a[1] for a in aggs], w1x, w1r, w1s,
        r1(p["nb_b1"]), p["nb_W2"], r1(p["nb_b2"]), p["nb_W3"],
        r1(p["nb_b3"]), r1(p["nb_lng"]), r1(p["nb_lnb"]))

    return (x_out, eo)
